# baseline (speedup 1.0000x reference)
import functools
import jax
import jax.numpy as jnp
from jax import lax
from jax.experimental import pallas as pl
from jax.experimental.pallas import tpu as pltpu
from jax.experimental.pallas import tpu_sc as plsc

_N = 32768
_D = 768
_E = 8
_NW = 32
_T = 4
_CH = _D // 16


def _sc_body(x_hbm, wrt_hbm, we_hbm, be_hbm, out_hbm, x_v, o_v, wrt_v, we_v, be_v):
    nc = 2
    wid = lax.axis_index("s") * nc + lax.axis_index("c")
    rows_per_w = _N // _NW
    base = wid * rows_per_w
    pltpu.sync_copy(wrt_hbm, wrt_v)
    pltpu.sync_copy(we_hbm, we_v)
    pltpu.sync_copy(be_hbm, be_v)
    lane = jnp.arange(16, dtype=jnp.int32)
    perms = [jnp.bitwise_xor(lane, jnp.int32(k)) for k in (8, 4, 2, 1)]

    def hsum(vec):
        for p in perms:
            vec = vec + jnp.take(vec, p)
        return vec

    def group(g, _):
        row0 = base + g * _T
        pltpu.sync_copy(x_hbm.at[pl.ds(row0, _T)], x_v)
        accs = [[jnp.zeros((16,), jnp.float32) for _ in range(_E)] for _ in range(_T)]
        for ci in range(_CH):
            xs = [x_v[t, pl.ds(ci * 16, 16)] for t in range(_T)]
            for e in range(_E):
                w = wrt_v[e, pl.ds(ci * 16, 16)]
                for t in range(_T):
                    accs[t][e] = accs[t][e] + xs[t] * w
        for t in range(_T):
            logits = [hsum(accs[t][e]) for e in range(_E)]
            best = logits[0]
            vrow = jnp.zeros((16,), jnp.int32)
            for e in range(1, _E):
                gt = logits[e] > best
                best = jnp.where(gt, logits[e], best)
                vrow = jnp.where(gt, jnp.full((16,), e, dtype=jnp.int32), vrow)
            vbase = vrow * jnp.int32(_D)
            for ci in range(_CH):
                idx = vbase + (lane + ci * 16)
                wv = plsc.load_gather(we_v, [idx])
                bv = plsc.load_gather(be_v, [idx])
                xc = x_v[t, pl.ds(ci * 16, 16)]
                o_v[t, pl.ds(ci * 16, 16)] = jnp.maximum(xc * wv + bv, 0.0)
        pltpu.sync_copy(o_v, out_hbm.at[pl.ds(row0, _T)])
        return 0

    lax.fori_loop(0, rows_per_w // _T, group, 0)


def kernel(x, w_router, w_expert, b_expert):
    mesh = plsc.VectorSubcoreMesh(core_axis_name="c", subcore_axis_name="s")
    wrt = w_router.T.reshape(_E, _D)
    k = functools.partial(
        pl.kernel,
        mesh=mesh,
        compiler_params=pltpu.CompilerParams(needs_layout_passes=False),
        out_type=jax.ShapeDtypeStruct((_N, _D), jnp.float32),
        scratch_types=[
            pltpu.VMEM((_T, _D), jnp.float32),
            pltpu.VMEM((_T, _D), jnp.float32),
            pltpu.VMEM((_E, _D), jnp.float32),
            pltpu.VMEM((_E * _D,), jnp.float32),
            pltpu.VMEM((_E * _D,), jnp.float32),
        ],
    )(_sc_body)
    return k(x, wrt, w_expert.reshape(-1), b_expert.reshape(-1))


# speedup vs baseline: 24.9765x; 24.9765x over previous
import jax
import jax.numpy as jnp
from jax.experimental import pallas as pl
from jax.experimental.pallas import tpu as pltpu

_BLOCK = 4096
_SUB = 1


def _body(x_ref, wr_ref, we_ref, be_ref, o_ref):
    block = x_ref.shape[0]
    sub = block // _SUB
    wr = wr_ref[...]
    we = we_ref[...]
    be = be_ref[...]
    for h in range(_SUB):
        x = x_ref[h * sub : (h + 1) * sub, :]
        logits = jnp.dot(x, wr, preferred_element_type=jnp.float32)
        values = jnp.argmax(logits, axis=-1)
        vb = jnp.broadcast_to(values[:, None], x.shape).astype(jnp.int32)
        w_tok = jnp.take_along_axis(we, vb, axis=0)
        b_tok = jnp.take_along_axis(be, vb, axis=0)
        o_ref[h * sub : (h + 1) * sub, :] = jnp.maximum(x * w_tok + b_tok, 0.0)


def kernel(x, w_router, w_expert, b_expert):
    n, d = x.shape
    e = w_router.shape[1]
    block = min(_BLOCK, n)
    return pl.pallas_call(
        _body,
        grid=(n // block,),
        in_specs=[
            pl.BlockSpec((block, d), lambda i: (i, 0)),
            pl.BlockSpec((d, e), lambda i: (0, 0)),
            pl.BlockSpec((e, d), lambda i: (0, 0)),
            pl.BlockSpec((e, d), lambda i: (0, 0)),
        ],
        out_specs=pl.BlockSpec((block, d), lambda i: (i, 0)),
        out_shape=jax.ShapeDtypeStruct((n, d), jnp.float32),
        compiler_params=pltpu.CompilerParams(
            dimension_semantics=("parallel",),
        ),
    )(x, w_router, w_expert, b_expert)
